# baseline (speedup 1.0000x reference)
import functools

import jax
import jax.numpy as jnp
from jax import lax
from jax.experimental import pallas as pl
from jax.experimental.pallas import tpu as pltpu
from jax.experimental.pallas import tpu_sc as plsc

N_ENT = 100000
N_REL = 200
D = 32
B = 1024
TEMP = 1.0
EPS = 1e-4

_NC = 2
_NS = 16
_NW = _NC * _NS
_BPW = B // _NW

_mesh = plsc.VectorSubcoreMesh(core_axis_name="c", subcore_axis_name="s")

_NCHUNK = (N_ENT + 127) // 128
_CPW = (_NCHUNK + _NW - 1) // _NW
_EPAD = _NCHUNK * 128


@functools.partial(
    pl.kernel,
    mesh=_mesh,
    compiler_params=pltpu.CompilerParams(needs_layout_passes=False),
    out_type=jax.ShapeDtypeStruct((_EPAD, 4 * D), jnp.float32),
    scratch_types=[
        pltpu.VMEM((2, D, 128), jnp.float32),
        pltpu.VMEM((2, D, 128), jnp.float32),
        pltpu.VMEM((2, D, 128), jnp.float32),
        pltpu.VMEM((2, D, 128), jnp.float32),
        pltpu.VMEM((128, 4 * D), jnp.float32),
        pltpu.SemaphoreType.DMA,
    ],
)
def _sc_relayout(pe_hbm, pi_hbm, fe_hbm, fi_hbm, out_hbm,
                 s0, s1, s2, s3, out_v, sem):
    wid = lax.axis_index("s") * _NC + lax.axis_index("c")
    slabs = (s0, s1, s2, s3)
    tabs = (pe_hbm, pi_hbm, fe_hbm, fi_hbm)
    iota = lax.iota(jnp.int32, 16)

    def fetch(cid, slot):
        for t in range(4):
            pltpu.async_copy(
                tabs[t].at[:, pl.ds(pl.multiple_of(cid * 128, 128), 128)],
                slabs[t].at[slot], sem)

    def drain(cid, slot):
        for t in range(4):
            pltpu.make_async_copy(
                tabs[t].at[:, pl.ds(pl.multiple_of(cid * 128, 128), 128)],
                slabs[t].at[slot], sem).wait()

    first = wid

    @pl.when(first < _NCHUNK)
    def _():
        fetch(first, 0)

    def round_body(k, carry):
        cid = wid + k * _NW

        @pl.when(cid < _NCHUNK)
        def _():
            slot = lax.rem(k, 2)
            nxt = cid + _NW

            @pl.when(nxt < _NCHUNK)
            def _():
                fetch(nxt, 1 - slot)

            drain(cid, slot)
            for t in range(4):
                slab = slabs[t]
                for d in range(D):
                    for e0 in range(0, 128, 16):
                        v = plsc.load_gather(
                            slab, [jnp.full((16,), slot, jnp.int32),
                                   jnp.full((16,), d, jnp.int32),
                                   e0 + iota])
                        plsc.store_scatter(
                            out_v, [e0 + iota,
                                    jnp.full((16,), t * D + d, jnp.int32)], v)
            pltpu.sync_copy(
                out_v, out_hbm.at[pl.ds(pl.multiple_of(cid * 128, 128), 128)])
        return carry

    lax.fori_loop(0, _CPW, round_body, 0)


@functools.partial(
    pl.kernel,
    mesh=_mesh,
    compiler_params=pltpu.CompilerParams(needs_layout_passes=False),
    out_type=jax.ShapeDtypeStruct((B,), jnp.float32),
    scratch_types=[
        pltpu.VMEM((_BPW,), jnp.int32),
        pltpu.VMEM((_BPW,), jnp.int32),
        pltpu.VMEM((_BPW,), jnp.int32),
        pltpu.VMEM((2 * _BPW,), jnp.int32),
        pltpu.VMEM((D, N_REL), jnp.float32),
        pltpu.VMEM((D, N_REL), jnp.float32),
        pltpu.VMEM((D, N_REL), jnp.float32),
        pltpu.VMEM((D, N_REL), jnp.float32),
        pltpu.VMEM((2 * _BPW, 4 * D), jnp.float32),
        pltpu.VMEM((2 * _BPW, 16), jnp.float32),
        pltpu.VMEM((_BPW,), jnp.float32),
        pltpu.SemaphoreType.DMA,
    ],
)
def _sc_scores(heads_hbm, rels_hbm, tails_hbm, ld_hbm, ent_hbm,
               prT_re_hbm, prT_im_hbm, frT_re_hbm, frT_im_hbm,
               out_hbm,
               h_v, r_v, t_v, ht_idx,
               pr_re, pr_im, fr_re, fr_im,
               ea, ld_segs, out_v, sem):
    wid = lax.axis_index("s") * _NC + lax.axis_index("c")
    base = wid * _BPW

    pltpu.sync_copy(heads_hbm.at[pl.ds(base, _BPW)], h_v)
    pltpu.sync_copy(rels_hbm.at[pl.ds(base, _BPW)], r_v)
    pltpu.sync_copy(tails_hbm.at[pl.ds(base, _BPW)], t_v)

    cps = [
        pltpu.async_copy(prT_re_hbm, pr_re, sem),
        pltpu.async_copy(prT_im_hbm, pr_im, sem),
        pltpu.async_copy(frT_re_hbm, fr_re, sem),
        pltpu.async_copy(frT_im_hbm, fr_im, sem),
    ]

    for j in range(_BPW):
        hc = h_v[pl.ds(16 * (j // 16), 16)]
        rc = r_v[pl.ds(16 * (j // 16), 16)]
        tc = t_v[pl.ds(16 * (j // 16), 16)]
        h = hc[j % 16]
        r = rc[j % 16]
        t = tc[j % 16]
        inv = r + 1 - 2 * (r % 2)
        cps.append(pltpu.async_copy(
            ld_hbm.at[r, pl.ds((h // 16) * 16, 16)], ld_segs.at[j], sem))
        cps.append(pltpu.async_copy(
            ld_hbm.at[inv, pl.ds((t // 16) * 16, 16)], ld_segs.at[_BPW + j], sem))

    for c in range(_BPW // 16):
        ht_idx[pl.ds(c * 16, 16)] = h_v[pl.ds(c * 16, 16)]
        ht_idx[pl.ds(_BPW + c * 16, 16)] = t_v[pl.ds(c * 16, 16)]
    cps.append(pltpu.async_copy(ent_hbm.at[ht_idx], ea, sem))
    for cp in cps:
        cp.wait()

    iota = lax.iota(jnp.int32, 16)
    zero = jnp.zeros((16,), jnp.float32)
    for half in range(_BPW // 16):
        row = half * 16 + iota
        rowt = row + _BPW
        rel_col = r_v[pl.ds(half * 16, 16)]

        def body(d, carry):
            acc_s, acc_p = carry
            dsp = jnp.full((16,), 0, jnp.int32) + d
            h_re = plsc.load_gather(ea, [row, dsp])
            h_im = plsc.load_gather(ea, [row, dsp + D])
            t_re = plsc.load_gather(ea, [rowt, dsp])
            t_im = plsc.load_gather(ea, [rowt, dsp + D])
            r_re = plsc.load_gather(pr_re, [dsp, rel_col])
            r_im = plsc.load_gather(pr_im, [dsp, rel_col])
            acc_s = acc_s + (h_re * r_re - h_im * r_im) * t_re \
                          + (h_re * r_im + h_im * r_re) * t_im
            g_re = plsc.load_gather(ea, [row, dsp + 2 * D])
            g_im = plsc.load_gather(ea, [row, dsp + 3 * D])
            u_re = plsc.load_gather(ea, [rowt, dsp + 2 * D])
            u_im = plsc.load_gather(ea, [rowt, dsp + 3 * D])
            q_re = plsc.load_gather(fr_re, [dsp, rel_col])
            q_im = plsc.load_gather(fr_im, [dsp, rel_col])
            acc_p = acc_p + (g_re * q_re - g_im * q_im) * u_re \
                          + (g_re * q_im + g_im * q_re) * u_im
            return acc_s, acc_p

        acc_s, acc_p = lax.fori_loop(0, D, body, (zero, zero))

        hc = h_v[pl.ds(half * 16, 16)]
        tc = t_v[pl.ds(half * 16, 16)]
        ld1 = plsc.load_gather(ld_segs, [row, hc % 16])
        ld2 = plsc.load_gather(ld_segs, [rowt, tc % 16])
        e = jnp.exp(TEMP * acc_s + jnp.maximum(ld1, ld2))
        scaled = jnp.clip(e, 0.0, 1.0 - EPS)
        pr_resp = jnp.where(acc_p > 0.0, 1.0, 0.0)
        out_v[pl.ds(half * 16, 16)] = (jnp.maximum(pr_resp, scaled) - 0.5) * 2.0

    pltpu.sync_copy(out_v, out_hbm.at[pl.ds(base, _BPW)])


def kernel(heads, rels, tails, logDelta,
           pred_ent_re, pred_ent_im, pred_rel_re, pred_rel_im,
           perf_ent_re, perf_ent_im, perf_rel_re, perf_rel_im):
    ent_all = _sc_relayout(pred_ent_re.T, pred_ent_im.T,
                           perf_ent_re.T, perf_ent_im.T)
    return _sc_scores(heads.astype(jnp.int32), rels.astype(jnp.int32),
                      tails.astype(jnp.int32), logDelta, ent_all,
                      pred_rel_re.T, pred_rel_im.T,
                      perf_rel_re.T, perf_rel_im.T)
